# baseline (speedup 1.0000x reference)
import functools

import jax
import jax.numpy as jnp
from jax import lax
from jax.experimental import pallas as pl
from jax.experimental.pallas import tpu as pltpu
from jax.experimental.pallas import tpu_sc as plsc

NUM_EXPERTS = 64
TOPK = 2
_GCHUNK = 128


def _sc_gather(tab, flat_ids):
    n = flat_ids.shape[0]
    info = plsc.get_sparse_core_info()
    num_workers = info.num_cores * info.num_subcores
    b_per_w = n // num_workers
    n_chunks = b_per_w // _GCHUNK
    ids3d = flat_ids.reshape(num_workers, n_chunks, _GCHUNK)
    mesh = plsc.VectorSubcoreMesh(core_axis_name="c", subcore_axis_name="s")

    @functools.partial(
        pl.kernel,
        mesh=mesh,
        compiler_params=pltpu.CompilerParams(use_tc_tiling_on_sc=False),
        out_type=jax.ShapeDtypeStruct((n,), jnp.int32),
        scratch_types=[
            pltpu.VMEM((n_chunks, _GCHUNK), jnp.int32),
            pltpu.VMEM((b_per_w,), jnp.int32),
            pltpu.SemaphoreType.DMA,
        ],
    )
    def gather_kernel(tab_hbm, ids_hbm, e01_hbm, idx_v, e_v, sem):
        wid = lax.axis_index("s") * info.num_cores + lax.axis_index("c")
        base = wid * b_per_w
        pltpu.sync_copy(ids_hbm.at[wid], idx_v)
        for j in range(n_chunks):
            pltpu.async_copy(
                tab_hbm.at[idx_v.at[j]],
                e_v.at[pl.ds(j * _GCHUNK, _GCHUNK)],
                sem,
            )
        for j in range(n_chunks):
            pltpu.make_async_copy(
                tab_hbm.at[idx_v.at[j]],
                e_v.at[pl.ds(j * _GCHUNK, _GCHUNK)],
                sem,
            ).wait()
        pltpu.sync_copy(e_v, e01_hbm.at[pl.ds(base, b_per_w)])

    return gather_kernel(tab, ids3d)


def _tc_expand(e01):
    g, _, b = e01.shape
    n = g * b

    def body(e_ref, probs_ref, map_ref):
        packed = jnp.broadcast_to(e_ref[0], (NUM_EXPERTS, b))
        iota = lax.broadcasted_iota(jnp.int32, (NUM_EXPERTS, b), 0)
        hit = (iota == (packed & 0xFFFF)) | (iota == (packed >> 16))
        probs_ref[...] = jnp.where(hit, jnp.float32(1.0 / TOPK), jnp.float32(0.0))
        map_ref[...] = hit.astype(jnp.int8)

    return pl.pallas_call(
        body,
        grid=(g,),
        in_specs=[pl.BlockSpec((1, 1, b), lambda i: (i, 0, 0))],
        out_specs=[
            pl.BlockSpec((NUM_EXPERTS, b), lambda i: (0, i)),
            pl.BlockSpec((NUM_EXPERTS, b), lambda i: (0, i)),
        ],
        out_shape=[
            jax.ShapeDtypeStruct((NUM_EXPERTS, n), jnp.float32),
            jax.ShapeDtypeStruct((NUM_EXPERTS, n), jnp.int8),
        ],
    )(e01)


_TC_BLOCK = 32768


def kernel(token_ids, tid2eid):
    flat_ids = token_ids.reshape(-1)
    n = flat_ids.shape[0]
    packed_tab = tid2eid[:, 0] | (tid2eid[:, 1] << 16)
    e01 = _sc_gather(packed_tab, flat_ids)
    g = n // _TC_BLOCK
    probs_t, map_t = _tc_expand(e01.reshape(g, 1, _TC_BLOCK))
    return probs_t.T, map_t.T.astype(bool)


# speedup vs baseline: 1.0249x; 1.0249x over previous
import functools

import jax
import jax.numpy as jnp
from jax import lax
from jax.experimental import pallas as pl
from jax.experimental.pallas import tpu as pltpu
from jax.experimental.pallas import tpu_sc as plsc

NUM_EXPERTS = 64
TOPK = 2
_GCHUNK = 128


def _sc_gather(tab, flat_ids):
    n = flat_ids.shape[0]
    info = plsc.get_sparse_core_info()
    num_workers = info.num_cores * info.num_subcores
    b_per_w = n // num_workers
    n_chunks = b_per_w // _GCHUNK
    ids3d = flat_ids.reshape(num_workers, n_chunks, _GCHUNK)
    mesh = plsc.VectorSubcoreMesh(core_axis_name="c", subcore_axis_name="s")

    @functools.partial(
        pl.kernel,
        mesh=mesh,
        compiler_params=pltpu.CompilerParams(use_tc_tiling_on_sc=False),
        out_type=jax.ShapeDtypeStruct((n,), jnp.int32),
        scratch_types=[
            pltpu.VMEM((n_chunks, _GCHUNK), jnp.int32),
            pltpu.VMEM((b_per_w,), jnp.int32),
            pltpu.SemaphoreType.DMA,
        ],
    )
    def gather_kernel(tab_hbm, ids_hbm, e01_hbm, idx_v, e_v, sem):
        wid = lax.axis_index("s") * info.num_cores + lax.axis_index("c")
        base = wid * b_per_w
        pltpu.sync_copy(ids_hbm.at[wid], idx_v)
        for j in range(n_chunks):
            pltpu.async_copy(
                tab_hbm.at[idx_v.at[j]],
                e_v.at[pl.ds(j * _GCHUNK, _GCHUNK)],
                sem,
            )
        for j in range(n_chunks):
            pltpu.make_async_copy(
                tab_hbm.at[idx_v.at[j]],
                e_v.at[pl.ds(j * _GCHUNK, _GCHUNK)],
                sem,
            ).wait()
        pltpu.sync_copy(e_v, e01_hbm.at[pl.ds(base, b_per_w)])

    return gather_kernel(tab, ids3d)


def _tc_expand(e01):
    g, _, b = e01.shape
    n = g * b

    def body(e_ref, probs_ref, map_ref):
        packed = jnp.broadcast_to(e_ref[0], (NUM_EXPERTS, b))
        iota = lax.broadcasted_iota(jnp.int32, (NUM_EXPERTS, b), 0)
        hit = (iota == (packed & 0xFFFF)) | (iota == (packed >> 16))
        probs_ref[...] = jnp.where(hit, jnp.float32(1.0 / TOPK), jnp.float32(0.0))
        map_ref[...] = hit.astype(jnp.int8)

    return pl.pallas_call(
        body,
        grid=(g,),
        in_specs=[pl.BlockSpec((1, 1, b), lambda i: (i, 0, 0))],
        out_specs=[
            pl.BlockSpec((NUM_EXPERTS, b), lambda i: (0, i)),
            pl.BlockSpec((NUM_EXPERTS, b), lambda i: (0, i)),
        ],
        out_shape=[
            jax.ShapeDtypeStruct((NUM_EXPERTS, n), jnp.float32),
            jax.ShapeDtypeStruct((NUM_EXPERTS, n), jnp.int8),
        ],
    )(e01)


_TC_BLOCK = 16384


def kernel(token_ids, tid2eid):
    flat_ids = token_ids.reshape(-1)
    n = flat_ids.shape[0]
    packed_tab = tid2eid[:, 0] | (tid2eid[:, 1] << 16)
    e01 = _sc_gather(packed_tab, flat_ids)
    g = n // _TC_BLOCK
    probs_t, map_t = _tc_expand(e01.reshape(g, 1, _TC_BLOCK))
    return probs_t.T, map_t.T.astype(bool)
